# baseline (speedup 1.0000x reference)
import functools

import jax
import jax.numpy as jnp
from jax import lax
from jax.experimental import pallas as pl
from jax.experimental.pallas import tpu as pltpu
from jax.experimental.pallas import tpu_sc as plsc

F = 128
F3 = 3 * F
ROW = F + F3
NSC = 2
NSUB = 16
NW = NSC * NSUB
C = 40


def _x_body(h_ref, v2_ref, w1_ref, b1_ref, w2_ref, b2_ref, xv_ref):
    a = h_ref[...] @ w1_ref[...] + b1_ref[...]
    a = a * jax.nn.sigmoid(a)
    xv_ref[:, :F3] = a @ w2_ref[...] + b2_ref[...]
    xv_ref[:, F3:] = v2_ref[...]


def _x_call(h2, V2, w1, b1r, w2, b2r, bn):
    n = h2.shape[0]
    return pl.pallas_call(
        _x_body,
        grid=(n // bn,),
        in_specs=[
            pl.BlockSpec((bn, F), lambda i: (i, 0)),
            pl.BlockSpec((bn, F3), lambda i: (i, 0)),
            pl.BlockSpec((F, F), lambda i: (0, 0)),
            pl.BlockSpec((1, F), lambda i: (0, 0)),
            pl.BlockSpec((F, F3), lambda i: (0, 0)),
            pl.BlockSpec((1, F3), lambda i: (0, 0)),
        ],
        out_specs=pl.BlockSpec((bn, 2 * F3), lambda i: (i, 0)),
        out_shape=jax.ShapeDtypeStruct((n, 2 * F3), jnp.float32),
    )(h2, V2, w1, b1r, w2, b2r)


def _edge_body(n_nodes, n_edges,
               xv_hbm, w_hbm, dir_hbm, ii_hbm, ij_hbm,
               plane_hbm, stag_hbm, sid_hbm,
               idxj_v, rows_xv, rows_w, acc, zrow, idv,
               ii_v, dir_v, head_v, prev_v, next_v, semA, semB):
    ep = n_edges // NW
    wid = lax.axis_index("s") * NSC + lax.axis_index("c")
    s0 = wid * ep

    zeros16 = jnp.zeros((16,), jnp.float32)
    for k in range(ROW // 16):
        acc[0, pl.ds(k * 16, 16)] = zeros16
        zrow[0, pl.ds(k * 16, 16)] = zeros16

    pltpu.sync_copy(ii_hbm.at[pl.ds(s0, 16)], head_v)

    @pl.when(wid > 0)
    def _():
        pltpu.sync_copy(ii_hbm.at[pl.ds(s0 - 8, 16)], prev_v)

    @pl.when(wid < NW - 1)
    def _():
        pltpu.sync_copy(ii_hbm.at[pl.ds(s0 + ep, 16)], next_v)

    first_node = head_v[pl.ds(0, 16)][0]
    prev_val = jnp.where(wid > 0, prev_v[pl.ds(0, 16)][7], -1)
    owner = (wid == 0) | (prev_val != first_node)
    next_first = jnp.where(wid < NW - 1, next_v[pl.ds(0, 16)][0], n_nodes)

    sidval = jnp.where(owner, 0, first_node).astype(jnp.float32)
    for k in range(F // 16):
        idv[0, pl.ds(k * 16, 16)] = jnp.broadcast_to(sidval, (16,))
    pltpu.sync_copy(zrow, stag_hbm.at[pl.ds(wid, 1)])
    pltpu.sync_copy(idv, sid_hbm.at[pl.ds(wid, 1)])

    def _zero_row(r, carry):
        pltpu.sync_copy(zrow, plane_hbm.at[pl.ds(r, 1)])
        return carry

    def _clear_acc():
        for k in range(ROW // 16):
            acc[0, pl.ds(k * 16, 16)] = jnp.zeros((16,), jnp.float32)

    nchunks = ep // C

    def _issue_a(ci):
        b = lax.rem(ci, 2)
        s = s0 + ci * C
        pltpu.async_copy(ii_hbm.at[pl.ds(s, C)], ii_v.at[pl.ds(b * 64, C)], semA)
        pltpu.async_copy(ij_hbm.at[pl.ds(s, C)], idxj_v.at[b], semA)
        pltpu.async_copy(dir_hbm.at[pl.ds(s * 3, C * 3)],
                         dir_v.at[pl.ds(b * 136, C * 3)], semA)
        pltpu.async_copy(w_hbm.at[pl.ds(s * F3, C * F3)], rows_w.at[b], semA)

    def _drain_a(ci):
        b = lax.rem(ci, 2)
        s = s0 + ci * C
        pltpu.make_async_copy(ii_hbm.at[pl.ds(s, C)], ii_v.at[pl.ds(b * 64, C)], semA).wait()
        pltpu.make_async_copy(ij_hbm.at[pl.ds(s, C)], idxj_v.at[b], semA).wait()
        pltpu.make_async_copy(dir_hbm.at[pl.ds(s * 3, C * 3)],
                              dir_v.at[pl.ds(b * 136, C * 3)], semA).wait()
        pltpu.make_async_copy(w_hbm.at[pl.ds(s * F3, C * F3)], rows_w.at[b], semA).wait()

    def _issue_b(ci):
        b = lax.rem(ci, 2)
        pltpu.async_copy(xv_hbm.at[idxj_v.at[b]], rows_xv.at[b], semB)

    def _drain_b(ci):
        b = lax.rem(ci, 2)
        pltpu.make_async_copy(xv_hbm.at[idxj_v.at[b]], rows_xv.at[b], semB).wait()

    def chunk_body(ci, carry):
        seg, cursor, done = carry
        b = lax.rem(ci, 2)
        _drain_b(ci)

        @pl.when(ci + 1 < nchunks)
        def _():
            _drain_a(ci + 1)
            _issue_b(ci + 1)

        def edge_body(l, carry2):
            seg2, cursor2, done2 = carry2
            n = ii_v[pl.ds(b * 64 + l, 16)][0]
            is_new = n != seg2
            is_stag = is_new & (done2 == 0) & jnp.logical_not(owner)
            is_plane = is_new & jnp.logical_not(is_stag)

            @pl.when(is_stag)
            def _():
                pltpu.sync_copy(acc, stag_hbm.at[pl.ds(wid, 1)])

            @pl.when(is_plane)
            def _():
                lax.fori_loop(cursor2 + 1, seg2, _zero_row, 0)
                pltpu.sync_copy(acc, plane_hbm.at[pl.ds(seg2, 1)])

            @pl.when(is_new)
            def _():
                _clear_acc()

            cursor3 = jnp.where(is_plane, seg2, cursor2)
            done3 = jnp.where(is_new, 1, done2)

            dvec = dir_v[pl.ds(b * 136 + l * 3, 16)]
            d0 = dvec[0]
            d1 = dvec[1]
            d2 = dvec[2]
            for g in range(2):
                ks = range(g * 4, g * 4 + 4)
                ld = {}
                for k in ks:
                    off = k * 16
                    ld[k] = (
                        rows_w[b, pl.ds(l * F3 + off, 16)],
                        rows_xv[b, l, pl.ds(off, 16)],
                        rows_w[b, pl.ds(l * F3 + F + off, 16)],
                        rows_xv[b, l, pl.ds(F + off, 16)],
                        rows_w[b, pl.ds(l * F3 + 2 * F + off, 16)],
                        rows_xv[b, l, pl.ds(2 * F + off, 16)],
                        rows_xv[b, l, pl.ds(F3 + off, 16)],
                        rows_xv[b, l, pl.ds(F3 + F + off, 16)],
                        rows_xv[b, l, pl.ds(F3 + 2 * F + off, 16)],
                    )
                res = {}
                for k in ks:
                    wh, xh, wr, xr, wv, xv, vj0, vj1, vj2 = ld[k]
                    dvr = wr * xr
                    dvv = wv * xv
                    res[k] = (wh * xh,
                              dvr * d0 + dvv * vj0,
                              dvr * d1 + dvv * vj1,
                              dvr * d2 + dvv * vj2)
                for k in ks:
                    off = k * 16
                    plsc.addupdate(acc.at[0, pl.ds(off, 16)], res[k][0])
                    plsc.addupdate(acc.at[0, pl.ds(F + off, 16)], res[k][1])
                    plsc.addupdate(acc.at[0, pl.ds(2 * F + off, 16)], res[k][2])
                    plsc.addupdate(acc.at[0, pl.ds(3 * F + off, 16)], res[k][3])
            return (n, cursor3, done3)

        out = lax.fori_loop(0, C, edge_body, (seg, cursor, done))

        @pl.when(ci + 2 < nchunks)
        def _():
            _issue_a(ci + 2)
        return out

    cursor0 = jnp.where(owner,
                        jnp.where(wid == 0, -1, first_node - 1),
                        first_node)
    _issue_a(jnp.int32(0))
    _drain_a(jnp.int32(0))
    _issue_b(jnp.int32(0))

    @pl.when(nchunks > 1)
    def _():
        _issue_a(jnp.int32(1))

    seg_f, cursor_f, done_f = lax.fori_loop(
        0, nchunks, chunk_body,
        (first_node, cursor0, jnp.int32(0)))

    is_stag = (done_f == 0) & jnp.logical_not(owner)

    @pl.when(is_stag)
    def _():
        pltpu.sync_copy(acc, stag_hbm.at[pl.ds(wid, 1)])

    @pl.when(jnp.logical_not(is_stag))
    def _():
        lax.fori_loop(cursor_f + 1, seg_f, _zero_row, 0)
        pltpu.sync_copy(acc, plane_hbm.at[pl.ds(seg_f, 1)])

    cursor_t = jnp.where(is_stag, cursor_f, seg_f)
    lax.fori_loop(cursor_t + 1, next_first, _zero_row, 0)


def _edge_call(xv, w3, dir_flat, ii, ij):
    n = xv.shape[0]
    e = ij.shape[0]
    mesh = plsc.VectorSubcoreMesh(core_axis_name="c", subcore_axis_name="s")
    kern = functools.partial(
        pl.kernel,
        out_type=(
            jax.ShapeDtypeStruct((n, ROW), jnp.float32),
            jax.ShapeDtypeStruct((NW, ROW), jnp.float32),
            jax.ShapeDtypeStruct((NW, F), jnp.float32),
        ),
        mesh=mesh,
        scratch_types=[
            pltpu.VMEM((2, C), jnp.int32),
            pltpu.VMEM((2, C, 2 * F3), jnp.float32),
            pltpu.VMEM((2, C * F3), jnp.float32),
            pltpu.VMEM((1, ROW), jnp.float32),
            pltpu.VMEM((1, ROW), jnp.float32),
            pltpu.VMEM((1, F), jnp.float32),
            pltpu.VMEM((144,), jnp.int32),
            pltpu.VMEM((2 * 136 + 16,), jnp.float32),
            pltpu.VMEM((16,), jnp.int32),
            pltpu.VMEM((16,), jnp.int32),
            pltpu.VMEM((16,), jnp.int32),
            pltpu.SemaphoreType.DMA,
            pltpu.SemaphoreType.DMA,
        ],
    )(functools.partial(_edge_body, n, e))
    return kern(xv, w3, dir_flat, ii, ij)


def _post_body(bn, h_ref, v_ref, pln_ref, st_ref, sid_ref,
               wmix_ref, wrem_ref, wfor_ref, wv_ref, g_ref, b_ref,
               q_ref, mu_ref):
    i = pl.program_id(0)
    rows = (lax.broadcasted_iota(jnp.int32, (bn, NW), 0) + i * bn
            ).astype(jnp.float32)
    ids = sid_ref[...][:, 0]
    onehot = (rows == ids[None, :]).astype(jnp.float32)
    plane = pln_ref[...] + onehot @ st_ref[...]
    dh = plane[:, :F]
    v = v_ref[...]

    wmix = wmix_ref[...]
    ctx = jnp.zeros((bn, F), jnp.float32)
    for c in range(3):
        vm = v[:, c * F:(c + 1) * F] @ wmix
        ctx = ctx + vm[:, :F] * vm[:, F:]

    q_pre = h_ref[...] + dh @ wrem_ref[...] + (dh @ wfor_ref[...]) * ctx
    mean = jnp.mean(q_pre, axis=-1, keepdims=True)
    var = jnp.mean((q_pre - mean) ** 2, axis=-1, keepdims=True)
    q_ref[...] = (q_pre - mean) * lax.rsqrt(var + 1e-5) * g_ref[...] + b_ref[...]

    s = jnp.sum(dh * wv_ref[...], axis=1, keepdims=True)
    mu_pre = v + plane[:, F:] + s * v
    m0 = mu_pre[:, :F]
    m1 = mu_pre[:, F:2 * F]
    m2 = mu_pre[:, 2 * F:]
    norm2 = m0 * m0 + m1 * m1 + m2 * m2
    rms = jnp.sqrt(jnp.mean(norm2, axis=-1, keepdims=True) + 1e-6)
    mu_ref[...] = mu_pre / rms


def _post_call(h2, v2, plane, stag, sid, wmix, wrem, wfor, wvr, gr, br, bn):
    n = h2.shape[0]
    return pl.pallas_call(
        functools.partial(_post_body, bn),
        grid=(n // bn,),
        in_specs=[
            pl.BlockSpec((bn, F), lambda i: (i, 0)),
            pl.BlockSpec((bn, F3), lambda i: (i, 0)),
            pl.BlockSpec((bn, ROW), lambda i: (i, 0)),
            pl.BlockSpec((NW, ROW), lambda i: (0, 0)),
            pl.BlockSpec((NW, F), lambda i: (0, 0)),
            pl.BlockSpec((F, 2 * F), lambda i: (0, 0)),
            pl.BlockSpec((F, F), lambda i: (0, 0)),
            pl.BlockSpec((F, F), lambda i: (0, 0)),
            pl.BlockSpec((1, F), lambda i: (0, 0)),
            pl.BlockSpec((1, F), lambda i: (0, 0)),
            pl.BlockSpec((1, F), lambda i: (0, 0)),
        ],
        out_specs=[
            pl.BlockSpec((bn, F), lambda i: (i, 0)),
            pl.BlockSpec((bn, F3), lambda i: (i, 0)),
        ],
        out_shape=[
            jax.ShapeDtypeStruct((n, F), jnp.float32),
            jax.ShapeDtypeStruct((n, F3), jnp.float32),
        ],
    )(h2, v2, plane, stag, sid, wmix, wrem, wfor, wvr, gr, br)


def kernel(h, v, H, V, Wij, dir_ij, idx_i, idx_j, n_atoms,
           W1, b1, W2, b2, Wmix, Wrem, Wfor, Wvrem, ln_g, ln_b):
    n = h.shape[0]
    e = Wij.shape[0]
    h2 = h.reshape(n, F)
    H2 = H.reshape(n, F)
    v2 = v.reshape(n, F3)
    V2 = V.reshape(n, F3)
    w3 = Wij.reshape(e * F3)
    dir_flat = dir_ij.reshape(e * 3)
    ii = idx_i.astype(jnp.int32)
    ij = idx_j.astype(jnp.int32)

    xv = _x_call(H2, V2, W1, b1.reshape(1, F), W2, b2.reshape(1, F3), bn=1000)
    plane, stag, sid = _edge_call(xv, w3, dir_flat, ii, ij)
    q2, mu2 = _post_call(h2, v2, plane, stag, sid, Wmix, Wrem, Wfor,
                         Wvrem.reshape(1, F), ln_g.reshape(1, F),
                         ln_b.reshape(1, F), bn=1000)
    return q2.reshape(n, 1, F), mu2.reshape(n, 3, F)


# speedup vs baseline: 1.0987x; 1.0987x over previous
import functools

import jax
import jax.numpy as jnp
from jax import lax
from jax.experimental import pallas as pl
from jax.experimental.pallas import tpu as pltpu
from jax.experimental.pallas import tpu_sc as plsc

F = 128
F3 = 3 * F
ROW = F + F3
NSC = 2
NSUB = 16
NW = NSC * NSUB
C = 40


def _x_body(h_ref, v2_ref, w1_ref, b1_ref, w2_ref, b2_ref, xv_ref):
    a = h_ref[...] @ w1_ref[...] + b1_ref[...]
    a = a * jax.nn.sigmoid(a)
    xv_ref[:, :F3] = a @ w2_ref[...] + b2_ref[...]
    xv_ref[:, F3:] = v2_ref[...]


def _x_call(h2, V2, w1, b1r, w2, b2r, bn):
    n = h2.shape[0]
    return pl.pallas_call(
        _x_body,
        grid=(n // bn,),
        in_specs=[
            pl.BlockSpec((bn, F), lambda i: (i, 0)),
            pl.BlockSpec((bn, F3), lambda i: (i, 0)),
            pl.BlockSpec((F, F), lambda i: (0, 0)),
            pl.BlockSpec((1, F), lambda i: (0, 0)),
            pl.BlockSpec((F, F3), lambda i: (0, 0)),
            pl.BlockSpec((1, F3), lambda i: (0, 0)),
        ],
        out_specs=pl.BlockSpec((bn, 2 * F3), lambda i: (i, 0)),
        out_shape=jax.ShapeDtypeStruct((n, 2 * F3), jnp.float32),
    )(h2, V2, w1, b1r, w2, b2r)


def _edge_body(n_nodes, n_edges,
               xv_hbm, w_hbm, dir_hbm, ii_hbm, ij_hbm,
               plane_hbm, stag_hbm, sid_hbm,
               idxj_v, rows_xv, rows_w, acc, zrow, idv,
               ii_v, dir_v, head_v, prev_v, next_v, semA, semB):
    ep = n_edges // NW
    wid = lax.axis_index("s") * NSC + lax.axis_index("c")
    s0 = wid * ep

    zeros16 = jnp.zeros((16,), jnp.float32)
    for k in range(ROW // 16):
        acc[0, pl.ds(k * 16, 16)] = zeros16
        zrow[0, pl.ds(k * 16, 16)] = zeros16

    pltpu.sync_copy(ii_hbm.at[pl.ds(s0, 16)], head_v)

    @pl.when(wid > 0)
    def _():
        pltpu.sync_copy(ii_hbm.at[pl.ds(s0 - 8, 16)], prev_v)

    @pl.when(wid < NW - 1)
    def _():
        pltpu.sync_copy(ii_hbm.at[pl.ds(s0 + ep, 16)], next_v)

    first_node = head_v[pl.ds(0, 16)][0]
    prev_val = jnp.where(wid > 0, prev_v[pl.ds(0, 16)][7], -1)
    owner = (wid == 0) | (prev_val != first_node)
    next_first = jnp.where(wid < NW - 1, next_v[pl.ds(0, 16)][0], n_nodes)

    sidval = jnp.where(owner, 0, first_node).astype(jnp.float32)
    for k in range(F // 16):
        idv[0, pl.ds(k * 16, 16)] = jnp.broadcast_to(sidval, (16,))
    pltpu.sync_copy(zrow, stag_hbm.at[pl.ds(wid, 1)])
    pltpu.sync_copy(idv, sid_hbm.at[pl.ds(wid, 1)])

    def _zero_row(r, carry):
        pltpu.sync_copy(zrow, plane_hbm.at[pl.ds(r, 1)])
        return carry

    def _clear_acc():
        for k in range(ROW // 16):
            acc[0, pl.ds(k * 16, 16)] = jnp.zeros((16,), jnp.float32)

    nchunks = ep // C

    def _issue_a(ci):
        b = lax.rem(ci, 2)
        s = s0 + ci * C
        pltpu.async_copy(ii_hbm.at[pl.ds(s, C)], ii_v.at[pl.ds(b * 64, C)], semA)
        pltpu.async_copy(ij_hbm.at[pl.ds(s, C)], idxj_v.at[b], semA)
        pltpu.async_copy(dir_hbm.at[pl.ds(s * 3, C * 3)],
                         dir_v.at[pl.ds(b * 136, C * 3)], semA)
        pltpu.async_copy(w_hbm.at[pl.ds(s * F3, C * F3)], rows_w.at[b], semA)

    def _drain_a(ci):
        b = lax.rem(ci, 2)
        s = s0 + ci * C
        pltpu.make_async_copy(ii_hbm.at[pl.ds(s, C)], ii_v.at[pl.ds(b * 64, C)], semA).wait()
        pltpu.make_async_copy(ij_hbm.at[pl.ds(s, C)], idxj_v.at[b], semA).wait()
        pltpu.make_async_copy(dir_hbm.at[pl.ds(s * 3, C * 3)],
                              dir_v.at[pl.ds(b * 136, C * 3)], semA).wait()
        pltpu.make_async_copy(w_hbm.at[pl.ds(s * F3, C * F3)], rows_w.at[b], semA).wait()

    def _issue_b(ci):
        b = lax.rem(ci, 2)
        pltpu.async_copy(xv_hbm.at[idxj_v.at[b]], rows_xv.at[b], semB)

    def _drain_b(ci):
        b = lax.rem(ci, 2)
        pltpu.make_async_copy(xv_hbm.at[idxj_v.at[b]], rows_xv.at[b], semB).wait()

    def chunk_body(ci, carry):
        seg, cursor, done = carry
        b = lax.rem(ci, 2)
        _drain_b(ci)

        @pl.when(ci + 1 < nchunks)
        def _():
            _drain_a(ci + 1)
            _issue_b(ci + 1)

        def _accum(l):
            dvec = dir_v[pl.ds(b * 136 + l * 3, 16)]
            d0 = dvec[0]
            d1 = dvec[1]
            d2 = dvec[2]
            for g in range(2):
                ks = range(g * 4, g * 4 + 4)
                ld = {}
                for k in ks:
                    off = k * 16
                    ld[k] = (
                        rows_w[b, pl.ds(l * F3 + off, 16)],
                        rows_xv[b, l, pl.ds(off, 16)],
                        rows_w[b, pl.ds(l * F3 + F + off, 16)],
                        rows_xv[b, l, pl.ds(F + off, 16)],
                        rows_w[b, pl.ds(l * F3 + 2 * F + off, 16)],
                        rows_xv[b, l, pl.ds(2 * F + off, 16)],
                        rows_xv[b, l, pl.ds(F3 + off, 16)],
                        rows_xv[b, l, pl.ds(F3 + F + off, 16)],
                        rows_xv[b, l, pl.ds(F3 + 2 * F + off, 16)],
                    )
                res = {}
                for k in ks:
                    wh, xh, wr, xr, wv, xv, vj0, vj1, vj2 = ld[k]
                    dvr = wr * xr
                    dvv = wv * xv
                    res[k] = (wh * xh,
                              dvr * d0 + dvv * vj0,
                              dvr * d1 + dvv * vj1,
                              dvr * d2 + dvv * vj2)
                for k in ks:
                    off = k * 16
                    plsc.addupdate(acc.at[0, pl.ds(off, 16)], res[k][0])
                    plsc.addupdate(acc.at[0, pl.ds(F + off, 16)], res[k][1])
                    plsc.addupdate(acc.at[0, pl.ds(2 * F + off, 16)], res[k][2])
                    plsc.addupdate(acc.at[0, pl.ds(3 * F + off, 16)], res[k][3])

        def edge_body(l, carry2):
            seg2, cursor2, done2 = carry2
            n = ii_v[pl.ds(b * 64 + l, 16)][0]
            is_new = n != seg2
            is_stag = is_new & (done2 == 0) & jnp.logical_not(owner)
            is_plane = is_new & jnp.logical_not(is_stag)

            @pl.when(is_stag)
            def _():
                pltpu.sync_copy(acc, stag_hbm.at[pl.ds(wid, 1)])

            @pl.when(is_plane)
            def _():
                lax.fori_loop(cursor2 + 1, seg2, _zero_row, 0)
                pltpu.sync_copy(acc, plane_hbm.at[pl.ds(seg2, 1)])

            @pl.when(is_new)
            def _():
                _clear_acc()

            cursor3 = jnp.where(is_plane, seg2, cursor2)
            done3 = jnp.where(is_new, 1, done2)
            _accum(l)
            return (n, cursor3, done3)

        def group_body(gi, carry2):
            seg2 = carry2[0]
            l0 = gi * 8
            n_z = ii_v[pl.ds(b * 64 + l0 + 7, 16)][0]
            fast = n_z == seg2

            @pl.when(fast)
            def _():
                for dl in range(8):
                    _accum(l0 + dl)

            return lax.fori_loop(l0, l0 + jnp.where(fast, 0, 8),
                                 edge_body, carry2)

        out = lax.fori_loop(0, C // 8, group_body, (seg, cursor, done))

        @pl.when(ci + 2 < nchunks)
        def _():
            _issue_a(ci + 2)
        return out

    cursor0 = jnp.where(owner,
                        jnp.where(wid == 0, -1, first_node - 1),
                        first_node)
    _issue_a(jnp.int32(0))
    _drain_a(jnp.int32(0))
    _issue_b(jnp.int32(0))

    @pl.when(nchunks > 1)
    def _():
        _issue_a(jnp.int32(1))

    seg_f, cursor_f, done_f = lax.fori_loop(
        0, nchunks, chunk_body,
        (first_node, cursor0, jnp.int32(0)))

    is_stag = (done_f == 0) & jnp.logical_not(owner)

    @pl.when(is_stag)
    def _():
        pltpu.sync_copy(acc, stag_hbm.at[pl.ds(wid, 1)])

    @pl.when(jnp.logical_not(is_stag))
    def _():
        lax.fori_loop(cursor_f + 1, seg_f, _zero_row, 0)
        pltpu.sync_copy(acc, plane_hbm.at[pl.ds(seg_f, 1)])

    cursor_t = jnp.where(is_stag, cursor_f, seg_f)
    lax.fori_loop(cursor_t + 1, next_first, _zero_row, 0)


def _edge_call(xv, w3, dir_flat, ii, ij):
    n = xv.shape[0]
    e = ij.shape[0]
    mesh = plsc.VectorSubcoreMesh(core_axis_name="c", subcore_axis_name="s")
    kern = functools.partial(
        pl.kernel,
        out_type=(
            jax.ShapeDtypeStruct((n, ROW), jnp.float32),
            jax.ShapeDtypeStruct((NW, ROW), jnp.float32),
            jax.ShapeDtypeStruct((NW, F), jnp.float32),
        ),
        mesh=mesh,
        scratch_types=[
            pltpu.VMEM((2, C), jnp.int32),
            pltpu.VMEM((2, C, 2 * F3), jnp.float32),
            pltpu.VMEM((2, C * F3), jnp.float32),
            pltpu.VMEM((1, ROW), jnp.float32),
            pltpu.VMEM((1, ROW), jnp.float32),
            pltpu.VMEM((1, F), jnp.float32),
            pltpu.VMEM((144,), jnp.int32),
            pltpu.VMEM((2 * 136 + 16,), jnp.float32),
            pltpu.VMEM((16,), jnp.int32),
            pltpu.VMEM((16,), jnp.int32),
            pltpu.VMEM((16,), jnp.int32),
            pltpu.SemaphoreType.DMA,
            pltpu.SemaphoreType.DMA,
        ],
    )(functools.partial(_edge_body, n, e))
    return kern(xv, w3, dir_flat, ii, ij)


def _post_body(bn, h_ref, v_ref, pln_ref, st_ref, sid_ref,
               wmix_ref, wrem_ref, wfor_ref, wv_ref, g_ref, b_ref,
               q_ref, mu_ref):
    i = pl.program_id(0)
    rows = (lax.broadcasted_iota(jnp.int32, (bn, NW), 0) + i * bn
            ).astype(jnp.float32)
    ids = sid_ref[...][:, 0]
    onehot = (rows == ids[None, :]).astype(jnp.float32)
    plane = pln_ref[...] + onehot @ st_ref[...]
    dh = plane[:, :F]
    v = v_ref[...]

    wmix = wmix_ref[...]
    ctx = jnp.zeros((bn, F), jnp.float32)
    for c in range(3):
        vm = v[:, c * F:(c + 1) * F] @ wmix
        ctx = ctx + vm[:, :F] * vm[:, F:]

    q_pre = h_ref[...] + dh @ wrem_ref[...] + (dh @ wfor_ref[...]) * ctx
    mean = jnp.mean(q_pre, axis=-1, keepdims=True)
    var = jnp.mean((q_pre - mean) ** 2, axis=-1, keepdims=True)
    q_ref[...] = (q_pre - mean) * lax.rsqrt(var + 1e-5) * g_ref[...] + b_ref[...]

    s = jnp.sum(dh * wv_ref[...], axis=1, keepdims=True)
    mu_pre = v + plane[:, F:] + s * v
    m0 = mu_pre[:, :F]
    m1 = mu_pre[:, F:2 * F]
    m2 = mu_pre[:, 2 * F:]
    norm2 = m0 * m0 + m1 * m1 + m2 * m2
    rms = jnp.sqrt(jnp.mean(norm2, axis=-1, keepdims=True) + 1e-6)
    mu_ref[...] = mu_pre / rms


def _post_call(h2, v2, plane, stag, sid, wmix, wrem, wfor, wvr, gr, br, bn):
    n = h2.shape[0]
    return pl.pallas_call(
        functools.partial(_post_body, bn),
        grid=(n // bn,),
        in_specs=[
            pl.BlockSpec((bn, F), lambda i: (i, 0)),
            pl.BlockSpec((bn, F3), lambda i: (i, 0)),
            pl.BlockSpec((bn, ROW), lambda i: (i, 0)),
            pl.BlockSpec((NW, ROW), lambda i: (0, 0)),
            pl.BlockSpec((NW, F), lambda i: (0, 0)),
            pl.BlockSpec((F, 2 * F), lambda i: (0, 0)),
            pl.BlockSpec((F, F), lambda i: (0, 0)),
            pl.BlockSpec((F, F), lambda i: (0, 0)),
            pl.BlockSpec((1, F), lambda i: (0, 0)),
            pl.BlockSpec((1, F), lambda i: (0, 0)),
            pl.BlockSpec((1, F), lambda i: (0, 0)),
        ],
        out_specs=[
            pl.BlockSpec((bn, F), lambda i: (i, 0)),
            pl.BlockSpec((bn, F3), lambda i: (i, 0)),
        ],
        out_shape=[
            jax.ShapeDtypeStruct((n, F), jnp.float32),
            jax.ShapeDtypeStruct((n, F3), jnp.float32),
        ],
    )(h2, v2, plane, stag, sid, wmix, wrem, wfor, wvr, gr, br)


def kernel(h, v, H, V, Wij, dir_ij, idx_i, idx_j, n_atoms,
           W1, b1, W2, b2, Wmix, Wrem, Wfor, Wvrem, ln_g, ln_b):
    n = h.shape[0]
    e = Wij.shape[0]
    h2 = h.reshape(n, F)
    H2 = H.reshape(n, F)
    v2 = v.reshape(n, F3)
    V2 = V.reshape(n, F3)
    w3 = Wij.reshape(e * F3)
    dir_flat = dir_ij.reshape(e * 3)
    ii = idx_i.astype(jnp.int32)
    ij = idx_j.astype(jnp.int32)

    xv = _x_call(H2, V2, W1, b1.reshape(1, F), W2, b2.reshape(1, F3), bn=1000)
    plane, stag, sid = _edge_call(xv, w3, dir_flat, ii, ij)
    q2, mu2 = _post_call(h2, v2, plane, stag, sid, Wmix, Wrem, Wfor,
                         Wvrem.reshape(1, F), ln_g.reshape(1, F),
                         ln_b.reshape(1, F), bn=1000)
    return q2.reshape(n, 1, F), mu2.reshape(n, 3, F)
